# baseline (speedup 1.0000x reference)
import functools

import jax
import jax.numpy as jnp
from jax import lax
from jax.experimental import pallas as pl
from jax.experimental.pallas import tpu as pltpu
from jax.experimental.pallas import tpu_sc as plsc

B, S, D = 4, 2048, 1024
NSRC = 2048
GD = 64
RR = 512
KEEP = S - RR
L = 16
NC, NS = 2, 16
TILES_PER_BATCH = (NC * NS) // B
PER_TILE = KEEP // TILES_PER_BATCH
CHX = 8
CHS = 8


def _tc_body(x_ref, w_ref, gn_ref, gn1_ref, rank_ref, sim_ref,
             gt_ref, simrow_ref):
    xb = x_ref[0]
    w = w_ref[...]
    gt_ref[...] = lax.dot_general(
        w, xb, (((1,), (1,)), ((), ())),
        precision=lax.Precision.HIGHEST,
        preferred_element_type=jnp.float32)

    CHN = 256
    cidx = lax.broadcasted_iota(jnp.int32, (GD, CHN), 1)
    last_lane = cidx == CHN - 1

    def nchunk(c, _):
        off = pl.multiple_of(c * CHN, CHN)
        noff = pl.multiple_of(
            jnp.where(c == S // CHN - 1, 0, (c + 1) * CHN), CHN)
        a = gt_ref[:, pl.ds(off, CHN)]
        na = gt_ref[:, pl.ds(noff, CHN)]
        b = jnp.where(last_lane,
                      pltpu.roll(na, CHN - 1, 1),
                      pltpu.roll(a, CHN - 1, 1))
        gna = jnp.sqrt(jnp.sum(a * a, axis=0, keepdims=True))
        gnb = jnp.sqrt(jnp.sum(b * b, axis=0, keepdims=True))
        au = a / jnp.maximum(gna, 1e-12)
        bu = b / jnp.maximum(gnb, 1e-12)
        simc = jnp.sum(au * bu, axis=0, keepdims=True)
        gn_ref[0, 0, pl.ds(off, CHN)] = gna[0]
        gn1_ref[0, 0, pl.ds(off, CHN)] = gnb[0]
        simrow_ref[0, pl.ds(off, CHN)] = simc[0]
        return 0

    lax.fori_loop(0, S // CHN, nchunk, 0)

    lidx = lax.broadcasted_iota(jnp.int32, (1, S), 1)
    simr = jnp.where(lidx == S - 1, -2.0, simrow_ref[...])
    sim_ref[...] = jnp.transpose(simr)

    CH = 128
    jdx = lax.broadcasted_iota(jnp.int32, (CH, S), 1)

    def chunk(c, _):
        si = sim_ref[pl.ds(c * CH, CH), :]
        gidx = c * CH + lax.broadcasted_iota(jnp.int32, (CH, S), 0)
        cmp = jnp.logical_or(
            simr > si, jnp.logical_and(simr == si, jdx < gidx))
        rk = jnp.sum(cmp.astype(jnp.int32), axis=1, keepdims=True)
        rank_ref[0, pl.ds(c * CH, CH), :] = rk
        return 0

    lax.fori_loop(0, S // CH, chunk, 0)


def _tc_stage(x, W):
    return pl.pallas_call(
        _tc_body,
        grid=(B,),
        in_specs=[
            pl.BlockSpec((1, S, D), lambda b: (b, 0, 0)),
            pl.BlockSpec((GD, D), lambda b: (0, 0)),
        ],
        out_specs=[
            pl.BlockSpec((1, 1, S), lambda b: (b, 0, 0)),
            pl.BlockSpec((1, 1, S), lambda b: (b, 0, 0)),
            pl.BlockSpec((1, S, 1), lambda b: (b, 0, 0)),
        ],
        out_shape=[
            jax.ShapeDtypeStruct((B, 1, S), jnp.float32),
            jax.ShapeDtypeStruct((B, 1, S), jnp.float32),
            jax.ShapeDtypeStruct((B, S, 1), jnp.int32),
        ],
        scratch_shapes=[pltpu.VMEM((S, 1), jnp.float32),
                        pltpu.VMEM((GD, S), jnp.float32),
                        pltpu.VMEM((1, S), jnp.float32)],
    )(x, W)


def _sc_body(x2d, src2d, pos2d, rank_h, gn_h, gn1_h,
             xm_h, sm_h, pm_h,
             rank_v, order_v,
             gn_v, gn1_v, pos_v,
             cidx_v, a_v, b_v, p_v,
             bufx1, bufs1,
             used_s, sel_s,
             sem0, sem1, sem4, sem5):
    cid = lax.axis_index("c")
    sid = lax.axis_index("s")
    wid = sid * NC + cid
    batch = wid // TILES_PER_BATCH
    lo = (wid % TILES_PER_BATCH) * PER_TILE
    gbase = batch * S
    obase = batch * KEEP + lo

    cpr = pltpu.async_copy(rank_h.at[batch], rank_v, sem0)
    cpg = pltpu.async_copy(gn_h.at[batch], gn_v, sem1)
    cpg1 = pltpu.async_copy(gn1_h.at[batch], gn1_v, sem4)
    cpp = pltpu.async_copy(pos2d.at[batch], pos_v, sem5)
    cpr.wait()
    cpg.wait()
    cpg1.wait()
    cpp.wait()

    iota16 = lax.broadcasted_iota(jnp.int32, (L,), 0)
    all16 = iota16 >= 0
    NW = S // 32

    @pl.loop(0, NW + 2)
    def _initw(wi):
        used_s[wi] = jnp.int32(0)
        sel_s[wi] = jnp.int32(0)

    @pl.loop(0, S // L)
    def _init(k):
        rv = rank_v[pl.ds(k * L, L)]
        plsc.store_scatter(order_v, [rv], k * L + iota16, mask=all16)

    one = jnp.int32(1)
    scope = jax.named_scope

    def gchunk(k, cnt):
        ov = order_v[pl.ds(k * L, L)]
        for j in range(L):
            i = ov[j]
            i1 = i + 1
            w1i = lax.shift_right_logical(i, 5)
            w2i = lax.shift_right_logical(i1, 5)
            b1 = i & 31
            b2 = i1 & 31
            w1 = used_s[w1i]
            w2 = used_s[w2i]
            same = w1i == w2i
            bit1 = lax.shift_right_logical(w1, b1) & 1
            bit2 = lax.shift_right_logical(w2, b2) & 1
            ok = (bit1 == 0) & (bit2 == 0) & (cnt < RR) & (i < S - 1)
            can = jnp.where(ok, one, jnp.int32(0))
            m1 = lax.shift_left(can, b1)
            m2 = lax.shift_left(can, b2)
            nw1 = w1 | m1 | jnp.where(same, m2, jnp.int32(0))
            nw2 = jnp.where(same, nw1, w2 | m2)
            used_s[w1i] = nw1
            used_s[w2i] = nw2
            sel_s[w1i] = sel_s[w1i] | m1
            cnt = cnt + can
        return cnt

    def gcond(state):
        k, cnt = state
        return jnp.logical_and(k < S // L, cnt < RR)

    def gstep(state):
        k, cnt = state
        return k + 1, gchunk(k, cnt)

    with scope("sc_greedy"):
        lax.while_loop(gcond, gstep, (jnp.int32(0), jnp.int32(0)))

    hi = lo + PER_TILE

    def cbody(k, running):
        tvec = k * L + iota16
        sh = (k & 1) * L
        w = lax.shift_right_logical(k, 1)
        cur = sel_s[w]
        praw = sel_s[jnp.maximum(w - 1, 0)]
        prev = jnp.where(w > 0, praw, jnp.int32(0))
        shifted = lax.shift_left(cur, 1) | lax.shift_right_logical(prev, 31)
        dw = jnp.full((L,), shifted, jnp.int32)
        sw = jnp.full((L,), cur, jnp.int32)
        dropc = lax.shift_right_logical(dw, iota16 + sh) & 1
        selc = lax.shift_right_logical(sw, iota16 + sh) & 1
        keepc = 1 - dropc
        cs = plsc.cumsum(keepc) + running
        s = cs - 1
        gnc = gn_v[pl.ds(k * L, L)]
        gn1c = gn1_v[pl.ds(k * L, L)]
        den = gnc + gn1c + jnp.float32(1e-8)
        selb = selc > 0
        af = jnp.where(selb, gnc / den, jnp.float32(1.0))
        bf = jnp.where(selb, gn1c / den, jnp.float32(0.0))
        mask = (dropc == 0) & (s >= lo) & (s < hi)
        srel = jnp.clip(s - lo, 0, PER_TILE - 1)
        pos1 = srel + (srel & ~jnp.int32(CHX - 1))
        plsc.store_scatter(cidx_v, [pos1], gbase + tvec, mask=mask)
        plsc.store_scatter(cidx_v, [pos1 + CHX], gbase + tvec + selc, mask=mask)
        plsc.store_scatter(a_v, [srel], af, mask=mask)
        plsc.store_scatter(b_v, [srel], bf, mask=mask)
        posc = pos_v[pl.ds(k * L, L)]
        plsc.store_scatter(p_v, [srel], posc, mask=mask)
        return jnp.max(cs)

    with scope("sc_compact"):
        lax.fori_loop(0, S // L, cbody, jnp.int32(0))

    pltpu.sync_copy(p_v.at[pl.ds(0, PER_TILE)], pm_h.at[pl.ds(obase, PER_TILE)])

    sems = (sem0, sem1)
    stsems = (sem4, sem5)
    NCHX = PER_TILE // CHX

    def issue_x(c, d):
        ii = cidx_v.at[pl.ds(c * 2 * CHX, 2 * CHX)]
        return pltpu.async_copy(x2d.at[ii], bufx1.at[d], sems[d])

    with scope("mark_x_start"):
        used_s[NW + 1] = jnp.int32(1)
    pend = [None, None]
    pend_st = [None, None]
    pend[0] = issue_x(0, 0)
    for c in range(NCHX):
        d = c & 1
        if c + 1 < NCHX:
            if pend_st[1 - d] is not None:
                pend_st[1 - d].wait()
                pend_st[1 - d] = None
            pend[1 - d] = issue_x(c + 1, 1 - d)
        pend[d].wait()
        base = c * CHX

        @pl.loop(0, CHX)
        def _row(j, base=base, d=d):
            av = a_v[pl.ds(base + j, L)][0]
            bv = b_v[pl.ds(base + j, L)][0]

            @pl.when(bv != 0.0)
            def _merge():
                @pl.loop(0, D // (L * 4))
                def _vec(v):
                    o = v * (L * 4)
                    r1s = [bufx1[d, j, pl.ds(o + k * L, L)] for k in range(4)]
                    r2s = [bufx1[d, CHX + j, pl.ds(o + k * L, L)]
                           for k in range(4)]
                    for k in range(4):
                        bufx1[d, j, pl.ds(o + k * L, L)] = (
                            av * r1s[k] + bv * r2s[k])

        pend_st[d] = pltpu.async_copy(
            bufx1.at[d, pl.ds(0, CHX)],
            xm_h.at[pl.ds(obase + base, CHX)], stsems[d])
    for d in range(2):
        if pend_st[d] is not None:
            pend_st[d].wait()

    NCHS = PER_TILE // CHS

    def issue_s(c, d):
        ii = cidx_v.at[pl.ds(c * 2 * CHS, 2 * CHS)]
        return pltpu.async_copy(src2d.at[ii], bufs1.at[d], sems[d])

    with scope("mark_s_start"):
        used_s[NW + 1] = jnp.int32(2)
    pend = [None, None]
    pend_st = [None, None]
    pend[0] = issue_s(0, 0)
    for c in range(NCHS):
        d = c & 1
        if c + 1 < NCHS:
            if pend_st[1 - d] is not None:
                pend_st[1 - d].wait()
                pend_st[1 - d] = None
            pend[1 - d] = issue_s(c + 1, 1 - d)
        pend[d].wait()
        base = c * CHS

        @pl.loop(0, CHS)
        def _row(j, base=base, d=d):
            iv = cidx_v[pl.ds(c * 2 * CHS + j, L)]
            dv = iv[CHS] - iv[0]

            @pl.when(dv != 0)
            def _merge():
                @pl.loop(0, NSRC // (L * 4))
                def _vec(v):
                    o = v * (L * 4)
                    r2s = [bufs1[d, CHS + j, pl.ds(o + k * L, L)]
                           for k in range(4)]
                    for k in range(4):
                        plsc.addupdate(
                            bufs1.at[d, j, pl.ds(o + k * L, L)], r2s[k])

        pend_st[d] = pltpu.async_copy(
            bufs1.at[d, pl.ds(0, CHS)],
            sm_h.at[pl.ds(obase + base, CHS)], stsems[d])
    for d in range(2):
        if pend_st[d] is not None:
            pend_st[d].wait()


def _sc_stage(x2d, src2d, pos2d, rank, gn, gn1):
    mesh = plsc.VectorSubcoreMesh(
        core_axis_name="c", subcore_axis_name="s",
        num_cores=NC, num_subcores=NS)
    f32, i32 = jnp.float32, jnp.int32
    return pl.kernel(
        _sc_body,
        out_type=[
            jax.ShapeDtypeStruct((B * KEEP, D), f32),
            jax.ShapeDtypeStruct((B * KEEP, NSRC), f32),
            jax.ShapeDtypeStruct((B * KEEP,), i32),
        ],
        mesh=mesh,
        compiler_params=pltpu.CompilerParams(needs_layout_passes=False),
        scratch_types=[
            pltpu.VMEM((S,), i32),
            pltpu.VMEM((S + 128,), i32),
            pltpu.VMEM((S,), f32),
            pltpu.VMEM((S,), f32),
            pltpu.VMEM((S,), i32),
            pltpu.VMEM((2 * PER_TILE + 128,), i32),
            pltpu.VMEM((PER_TILE + 128,), f32),
            pltpu.VMEM((PER_TILE + 128,), f32),
            pltpu.VMEM((PER_TILE,), i32),
            pltpu.VMEM((2, 2 * CHX, D), f32),
            pltpu.VMEM((2, 2 * CHS, NSRC), f32),
            pltpu.SMEM((S // 32 + 2,), i32),
            pltpu.SMEM((S // 32 + 2,), i32),
            pltpu.SemaphoreType.DMA,
            pltpu.SemaphoreType.DMA,
            pltpu.SemaphoreType.DMA,
            pltpu.SemaphoreType.DMA,
        ],
    )(x2d, src2d, pos2d, rank, gn, gn1)


def kernel(x, source, position_ids, r, W):
    gn, gn1, rank = _tc_stage(x, W)
    gn = gn.reshape(B, S)
    gn1 = gn1.reshape(B, S)
    rank = rank.reshape(B, S)
    x2d = x.reshape(B * S, D)
    src2d = source.reshape(B * S, NSRC)
    xm, sm, pm = _sc_stage(x2d, src2d, position_ids, rank, gn, gn1)
    return (xm.reshape(B, KEEP, D),
            sm.reshape(B, KEEP, NSRC),
            pm.reshape(B, KEEP))


# speedup vs baseline: 1.0227x; 1.0227x over previous
import functools

import jax
import jax.numpy as jnp
from jax import lax
from jax.experimental import pallas as pl
from jax.experimental.pallas import tpu as pltpu
from jax.experimental.pallas import tpu_sc as plsc

B, S, D = 4, 2048, 1024
NSRC = 2048
GD = 64
RR = 512
KEEP = S - RR
L = 16
NC, NS = 2, 16
TILES_PER_BATCH = (NC * NS) // B
PER_TILE = KEEP // TILES_PER_BATCH
CHX = 8
CHS = 8


def _tc_body(x_ref, w_ref, gn_ref, gn1_ref, rank_ref, sim_ref):
    xb = x_ref[0]
    w = w_ref[...]
    gT = lax.dot_general(
        w, xb, (((1,), (1,)), ((), ())),
        precision=lax.Precision.HIGHEST,
        preferred_element_type=jnp.float32)
    gsq = jnp.sum(gT * gT, axis=0, keepdims=True)
    gnr = jnp.sqrt(gsq)
    gu = gT / jnp.maximum(gnr, 1e-12)
    gu1 = pltpu.roll(gu, S - 1, 1)
    simr = jnp.sum(gu * gu1, axis=0, keepdims=True)
    lidx = lax.broadcasted_iota(jnp.int32, (1, S), 1)
    simr = jnp.where(lidx == S - 1, -2.0, simr)

    gn_ref[0] = gnr
    gn1_ref[0] = pltpu.roll(gnr, S - 1, 1)
    sim_ref[...] = jnp.transpose(simr)

    CH = 128
    jdx = lax.broadcasted_iota(jnp.int32, (CH, S), 1)

    def chunk(c, _):
        si = sim_ref[pl.ds(c * CH, CH), :]
        gidx = c * CH + lax.broadcasted_iota(jnp.int32, (CH, S), 0)
        cmp = jnp.logical_or(
            simr > si, jnp.logical_and(simr == si, jdx < gidx))
        rk = jnp.sum(cmp.astype(jnp.int32), axis=1, keepdims=True)
        rank_ref[0, pl.ds(c * CH, CH), :] = rk
        return 0

    lax.fori_loop(0, S // CH, chunk, 0)


def _tc_stage(x, W):
    return pl.pallas_call(
        _tc_body,
        grid=(B,),
        in_specs=[
            pl.BlockSpec((1, S, D), lambda b: (b, 0, 0)),
            pl.BlockSpec((GD, D), lambda b: (0, 0)),
        ],
        out_specs=[
            pl.BlockSpec((1, 1, S), lambda b: (b, 0, 0)),
            pl.BlockSpec((1, 1, S), lambda b: (b, 0, 0)),
            pl.BlockSpec((1, S, 1), lambda b: (b, 0, 0)),
        ],
        out_shape=[
            jax.ShapeDtypeStruct((B, 1, S), jnp.float32),
            jax.ShapeDtypeStruct((B, 1, S), jnp.float32),
            jax.ShapeDtypeStruct((B, S, 1), jnp.int32),
        ],
        scratch_shapes=[pltpu.VMEM((S, 1), jnp.float32)],
    )(x, W)


def _sc_body(x2d, src2d, pos2d, rank_h, gn_h, gn1_h,
             xm_h, sm_h, pm_h,
             rank_v, order_v,
             gn_v, gn1_v, pos_v,
             cidx_v, a_v, b_v, p_v,
             bufx1, bufs1,
             used_s, sel_s,
             sem0, sem1, sem4, sem5):
    cid = lax.axis_index("c")
    sid = lax.axis_index("s")
    wid = sid * NC + cid
    batch = wid // TILES_PER_BATCH
    lo = (wid % TILES_PER_BATCH) * PER_TILE
    gbase = batch * S
    obase = batch * KEEP + lo

    cpr = pltpu.async_copy(rank_h.at[batch], rank_v, sem0)
    cpg = pltpu.async_copy(gn_h.at[batch], gn_v, sem1)
    cpg1 = pltpu.async_copy(gn1_h.at[batch], gn1_v, sem4)
    cpp = pltpu.async_copy(pos2d.at[batch], pos_v, sem5)
    cpr.wait()
    cpg.wait()
    cpg1.wait()
    cpp.wait()

    iota16 = lax.broadcasted_iota(jnp.int32, (L,), 0)
    all16 = iota16 >= 0
    NW = S // 32

    @pl.loop(0, NW + 2)
    def _initw(wi):
        used_s[wi] = jnp.int32(0)
        sel_s[wi] = jnp.int32(0)

    @pl.loop(0, S // L)
    def _init(k):
        rv = rank_v[pl.ds(k * L, L)]
        plsc.store_scatter(order_v, [rv], k * L + iota16, mask=all16)

    one = jnp.int32(1)
    scope = jax.named_scope

    def gchunk(k, cnt):
        ov = order_v[pl.ds(k * L, L)]
        for j in range(L):
            i = ov[j]
            i1 = i + 1
            w1i = lax.shift_right_logical(i, 5)
            w2i = lax.shift_right_logical(i1, 5)
            b1 = i & 31
            b2 = i1 & 31
            w1 = used_s[w1i]
            w2 = used_s[w2i]
            same = w1i == w2i
            bit1 = lax.shift_right_logical(w1, b1) & 1
            bit2 = lax.shift_right_logical(w2, b2) & 1
            ok = (bit1 == 0) & (bit2 == 0) & (cnt < RR) & (i < S - 1)
            can = jnp.where(ok, one, jnp.int32(0))
            m1 = lax.shift_left(can, b1)
            m2 = lax.shift_left(can, b2)
            nw1 = w1 | m1 | jnp.where(same, m2, jnp.int32(0))
            nw2 = jnp.where(same, nw1, w2 | m2)
            used_s[w1i] = nw1
            used_s[w2i] = nw2
            sel_s[w1i] = sel_s[w1i] | m1
            cnt = cnt + can
        return cnt

    def gcond(state):
        k, cnt = state
        return jnp.logical_and(k < S // L, cnt < RR)

    def gstep(state):
        k, cnt = state
        return k + 1, gchunk(k, cnt)

    with scope("sc_greedy"):
        lax.while_loop(gcond, gstep, (jnp.int32(0), jnp.int32(0)))

    hi = lo + PER_TILE

    def cbody(k, running):
        tvec = k * L + iota16
        sh = (k & 1) * L
        w = lax.shift_right_logical(k, 1)
        cur = sel_s[w]
        praw = sel_s[jnp.maximum(w - 1, 0)]
        prev = jnp.where(w > 0, praw, jnp.int32(0))
        shifted = lax.shift_left(cur, 1) | lax.shift_right_logical(prev, 31)
        dw = jnp.full((L,), shifted, jnp.int32)
        sw = jnp.full((L,), cur, jnp.int32)
        dropc = lax.shift_right_logical(dw, iota16 + sh) & 1
        selc = lax.shift_right_logical(sw, iota16 + sh) & 1
        keepc = 1 - dropc
        cs = plsc.cumsum(keepc) + running
        s = cs - 1
        gnc = gn_v[pl.ds(k * L, L)]
        gn1c = gn1_v[pl.ds(k * L, L)]
        den = gnc + gn1c + jnp.float32(1e-8)
        selb = selc > 0
        af = jnp.where(selb, gnc / den, jnp.float32(1.0))
        bf = jnp.where(selb, gn1c / den, jnp.float32(0.0))
        mask = (dropc == 0) & (s >= lo) & (s < hi)
        srel = jnp.clip(s - lo, 0, PER_TILE - 1)
        pos1 = srel + (srel & ~jnp.int32(CHX - 1))
        plsc.store_scatter(cidx_v, [pos1], gbase + tvec, mask=mask)
        plsc.store_scatter(cidx_v, [pos1 + CHX], gbase + tvec + selc, mask=mask)
        plsc.store_scatter(a_v, [srel], af, mask=mask)
        plsc.store_scatter(b_v, [srel], bf, mask=mask)
        posc = pos_v[pl.ds(k * L, L)]
        plsc.store_scatter(p_v, [srel], posc, mask=mask)
        return jnp.max(cs)

    with scope("sc_compact"):
        lax.fori_loop(0, S // L, cbody, jnp.int32(0))

    pltpu.sync_copy(p_v.at[pl.ds(0, PER_TILE)], pm_h.at[pl.ds(obase, PER_TILE)])

    sems = (sem0, sem1)
    stsems = (sem4, sem5)
    NCHX = PER_TILE // CHX

    def issue_x(c, d):
        ii = cidx_v.at[pl.ds(c * 2 * CHX, 2 * CHX)]
        return pltpu.async_copy(x2d.at[ii], bufx1.at[d], sems[d])

    with scope("mark_x_start"):
        used_s[NW + 1] = jnp.int32(1)
    pend = [None, None]
    pend_st = [None, None]
    pend[0] = issue_x(0, 0)
    for c in range(NCHX):
        d = c & 1
        if c + 1 < NCHX:
            if pend_st[1 - d] is not None:
                pend_st[1 - d].wait()
                pend_st[1 - d] = None
            pend[1 - d] = issue_x(c + 1, 1 - d)
        pend[d].wait()
        base = c * CHX

        @pl.loop(0, CHX)
        def _row(j, base=base, d=d):
            av = a_v[pl.ds(base + j, L)][0]
            bv = b_v[pl.ds(base + j, L)][0]

            @pl.when(bv != 0.0)
            def _merge():
                @pl.loop(0, D // (L * 4))
                def _vec(v):
                    o = v * (L * 4)
                    r1s = [bufx1[d, j, pl.ds(o + k * L, L)] for k in range(4)]
                    r2s = [bufx1[d, CHX + j, pl.ds(o + k * L, L)]
                           for k in range(4)]
                    for k in range(4):
                        bufx1[d, j, pl.ds(o + k * L, L)] = (
                            av * r1s[k] + bv * r2s[k])

        pend_st[d] = pltpu.async_copy(
            bufx1.at[d, pl.ds(0, CHX)],
            xm_h.at[pl.ds(obase + base, CHX)], stsems[d])
    for d in range(2):
        if pend_st[d] is not None:
            pend_st[d].wait()

    NCHS = PER_TILE // CHS

    def issue_s(c, d):
        ii = cidx_v.at[pl.ds(c * 2 * CHS, 2 * CHS)]
        return pltpu.async_copy(src2d.at[ii], bufs1.at[d], sems[d])

    with scope("mark_s_start"):
        used_s[NW + 1] = jnp.int32(2)
    pend = [None, None]
    pend_st = [None, None]
    pend[0] = issue_s(0, 0)
    for c in range(NCHS):
        d = c & 1
        if c + 1 < NCHS:
            if pend_st[1 - d] is not None:
                pend_st[1 - d].wait()
                pend_st[1 - d] = None
            pend[1 - d] = issue_s(c + 1, 1 - d)
        pend[d].wait()
        base = c * CHS

        @pl.loop(0, CHS)
        def _row(j, base=base, d=d):
            iv = cidx_v[pl.ds(c * 2 * CHS + j, L)]
            dv = iv[CHS] - iv[0]

            @pl.when(dv != 0)
            def _merge():
                @pl.loop(0, NSRC // (L * 4))
                def _vec(v):
                    o = v * (L * 4)
                    r2s = [bufs1[d, CHS + j, pl.ds(o + k * L, L)]
                           for k in range(4)]
                    for k in range(4):
                        plsc.addupdate(
                            bufs1.at[d, j, pl.ds(o + k * L, L)], r2s[k])

        pend_st[d] = pltpu.async_copy(
            bufs1.at[d, pl.ds(0, CHS)],
            sm_h.at[pl.ds(obase + base, CHS)], stsems[d])
    for d in range(2):
        if pend_st[d] is not None:
            pend_st[d].wait()


def _sc_stage(x2d, src2d, pos2d, rank, gn, gn1):
    mesh = plsc.VectorSubcoreMesh(
        core_axis_name="c", subcore_axis_name="s",
        num_cores=NC, num_subcores=NS)
    f32, i32 = jnp.float32, jnp.int32
    return pl.kernel(
        _sc_body,
        out_type=[
            jax.ShapeDtypeStruct((B * KEEP, D), f32),
            jax.ShapeDtypeStruct((B * KEEP, NSRC), f32),
            jax.ShapeDtypeStruct((B * KEEP,), i32),
        ],
        mesh=mesh,
        compiler_params=pltpu.CompilerParams(needs_layout_passes=False),
        scratch_types=[
            pltpu.VMEM((S,), i32),
            pltpu.VMEM((S + 128,), i32),
            pltpu.VMEM((S,), f32),
            pltpu.VMEM((S,), f32),
            pltpu.VMEM((S,), i32),
            pltpu.VMEM((2 * PER_TILE + 128,), i32),
            pltpu.VMEM((PER_TILE + 128,), f32),
            pltpu.VMEM((PER_TILE + 128,), f32),
            pltpu.VMEM((PER_TILE,), i32),
            pltpu.VMEM((2, 2 * CHX, D), f32),
            pltpu.VMEM((2, 2 * CHS, NSRC), f32),
            pltpu.SMEM((S // 32 + 2,), i32),
            pltpu.SMEM((S // 32 + 2,), i32),
            pltpu.SemaphoreType.DMA,
            pltpu.SemaphoreType.DMA,
            pltpu.SemaphoreType.DMA,
            pltpu.SemaphoreType.DMA,
        ],
    )(x2d, src2d, pos2d, rank, gn, gn1)


def kernel(x, source, position_ids, r, W):
    gn, gn1, rank = _tc_stage(x, W)
    gn = gn.reshape(B, S)
    gn1 = gn1.reshape(B, S)
    rank = rank.reshape(B, S)
    x2d = x.reshape(B * S, D)
    src2d = source.reshape(B * S, NSRC)
    xm, sm, pm = _sc_stage(x2d, src2d, position_ids, rank, gn, gn1)
    return (xm.reshape(B, KEEP, D),
            sm.reshape(B, KEEP, NSRC),
            pm.reshape(B, KEEP))
